# baseline (device time: 41011 ns/iter reference)
import jax
import jax.numpy as jnp
from jax import lax
from jax.experimental import pallas as pl
from jax.experimental.pallas import tpu as pltpu

N_DEV = 4


def kernel(x, w_mat):
    m_per, k = x.shape
    n = w_mat.shape[1]
    n_per = n // N_DEV
    m = N_DEV * m_per

    def body(x_ref, w_ref, out_ref, comm_ref, send_sems, recv_sems):
        my = lax.axis_index("i")

        barrier_sem = pltpu.get_barrier_semaphore()
        for d in range(1, N_DEV):
            pl.semaphore_signal(
                barrier_sem, inc=1,
                device_id=((my + d) % N_DEV,),
                device_id_type=pl.DeviceIdType.MESH,
            )
        pl.semaphore_wait(barrier_sem, N_DEV - 1)

        xv = x_ref[:, :]

        def make_rdma(d):
            tgt = (my + d) % N_DEV
            return pltpu.make_async_remote_copy(
                src_ref=comm_ref.at[d - 1],
                dst_ref=out_ref.at[pl.ds(my * m_per, m_per), :],
                send_sem=send_sems.at[d - 1],
                recv_sem=recv_sems.at[d - 1],
                device_id=(tgt,),
                device_id_type=pl.DeviceIdType.MESH,
            )

        def chunk(tgt):
            w_blk = w_ref[:, pl.ds(tgt * n_per, n_per)]
            yblk = jnp.dot(xv, w_blk, preferred_element_type=jnp.float32)
            return yblk * jax.nn.sigmoid(yblk)

        rdmas = {}
        for d in (1, 3):
            comm_ref[d - 1, :, :] = chunk((my + d) % N_DEV)
            rdmas[d] = make_rdma(d)
            rdmas[d].start()

        comm_ref[1, :, :] = chunk((my + 2) % N_DEV)
        w_blk = w_ref[:, pl.ds(my * n_per, n_per)]
        yblk = jnp.dot(xv, w_blk, preferred_element_type=jnp.float32)
        out_ref[pl.ds(my * m_per, m_per), :] = yblk * jax.nn.sigmoid(yblk)

        rdmas[1].wait_send()
        rdmas[3].wait_send()
        rdmas[2] = make_rdma(2)
        rdmas[2].start()

        rdmas[1].wait_recv()
        rdmas[3].wait_recv()
        rdmas[2].wait_send()
        rdmas[2].wait_recv()

    return pl.pallas_call(
        body,
        out_shape=jax.ShapeDtypeStruct((m, n_per), jnp.float32),
        in_specs=[
            pl.BlockSpec(memory_space=pltpu.VMEM),
            pl.BlockSpec(memory_space=pltpu.VMEM),
        ],
        out_specs=pl.BlockSpec(memory_space=pltpu.VMEM),
        scratch_shapes=[
            pltpu.VMEM((N_DEV - 1, m_per, n_per), jnp.float32),
            pltpu.SemaphoreType.DMA((N_DEV - 1,)),
            pltpu.SemaphoreType.DMA((N_DEV - 1,)),
        ],
        compiler_params=pltpu.CompilerParams(collective_id=0),
    )(x, w_mat)


# device time: 38921 ns/iter; 1.0537x vs baseline; 1.0537x over previous
import jax
import jax.numpy as jnp
from jax import lax
from jax.experimental import pallas as pl
from jax.experimental.pallas import tpu as pltpu

N_DEV = 4


def kernel(x, w_mat):
    m_per, k = x.shape
    n = w_mat.shape[1]
    n_per = n // N_DEV
    m = N_DEV * m_per

    def body(x_ref, w_ref, out_ref, comm_ref, send_sems, recv_sems):
        my = lax.axis_index("i")

        barrier_sem = pltpu.get_barrier_semaphore()
        for d in range(1, N_DEV):
            pl.semaphore_signal(
                barrier_sem, inc=1,
                device_id=((my + d) % N_DEV,),
                device_id_type=pl.DeviceIdType.MESH,
            )
        pl.semaphore_wait(barrier_sem, N_DEV - 1)

        xv = x_ref[:, :]

        def make_rdma(d):
            tgt = (my + d) % N_DEV
            return pltpu.make_async_remote_copy(
                src_ref=comm_ref.at[d - 1],
                dst_ref=out_ref.at[pl.ds(my * m_per, m_per), :],
                send_sem=send_sems.at[d - 1],
                recv_sem=recv_sems.at[d - 1],
                device_id=(tgt,),
                device_id_type=pl.DeviceIdType.MESH,
            )

        def chunk(tgt):
            w_blk = w_ref[:, pl.ds(tgt * n_per, n_per)]
            yblk = jnp.dot(xv, w_blk, preferred_element_type=jnp.float32)
            return yblk * jax.nn.sigmoid(yblk)

        rdmas = []
        for d in (2, 1, 3):
            comm_ref[d - 1, :, :] = chunk((my + d) % N_DEV)
            rdma = make_rdma(d)
            rdma.start()
            rdmas.append(rdma)

        w_blk = w_ref[:, pl.ds(my * n_per, n_per)]
        yblk = jnp.dot(xv, w_blk, preferred_element_type=jnp.float32)
        out_ref[pl.ds(my * m_per, m_per), :] = yblk * jax.nn.sigmoid(yblk)

        for rdma in rdmas:
            rdma.wait_send()
            rdma.wait_recv()

    return pl.pallas_call(
        body,
        out_shape=jax.ShapeDtypeStruct((m, n_per), jnp.float32),
        in_specs=[
            pl.BlockSpec(memory_space=pltpu.VMEM),
            pl.BlockSpec(memory_space=pltpu.VMEM),
        ],
        out_specs=pl.BlockSpec(memory_space=pltpu.VMEM),
        scratch_shapes=[
            pltpu.VMEM((N_DEV - 1, m_per, n_per), jnp.float32),
            pltpu.SemaphoreType.DMA((N_DEV - 1,)),
            pltpu.SemaphoreType.DMA((N_DEV - 1,)),
        ],
        compiler_params=pltpu.CompilerParams(collective_id=0),
    )(x, w_mat)


# device time: 27946 ns/iter; 1.4675x vs baseline; 1.3927x over previous
import jax
import jax.numpy as jnp
from jax import lax
from jax.experimental import pallas as pl
from jax.experimental.pallas import tpu as pltpu

N_DEV = 4


def kernel(x, w_mat):
    m_per, k = x.shape
    n = w_mat.shape[1]
    n_per = n // N_DEV
    m = N_DEV * m_per

    def body(x_ref, w_ref, out_ref, comm_ref, recv_ref, send_sems, recv_sems):
        my = lax.axis_index("i")

        barrier_sem = pltpu.get_barrier_semaphore()
        for d in range(1, N_DEV):
            pl.semaphore_signal(
                barrier_sem, inc=1,
                device_id=((my + d) % N_DEV,),
                device_id_type=pl.DeviceIdType.MESH,
            )
        pl.semaphore_wait(barrier_sem, N_DEV - 1)

        xv = x_ref[:, :]

        def make_rdma(d):
            tgt = (my + d) % N_DEV
            return pltpu.make_async_remote_copy(
                src_ref=comm_ref.at[d - 1],
                dst_ref=recv_ref.at[d - 1],
                send_sem=send_sems.at[d - 1],
                recv_sem=recv_sems.at[d - 1],
                device_id=(tgt,),
                device_id_type=pl.DeviceIdType.MESH,
            )

        def chunk(tgt):
            w_blk = w_ref[:, pl.ds(tgt * n_per, n_per)]
            yblk = jnp.dot(xv, w_blk, preferred_element_type=jnp.float32)
            return yblk * jax.nn.sigmoid(yblk)

        rdmas = {}
        for d in (2, 1, 3):
            comm_ref[d - 1, :, :] = chunk((my + d) % N_DEV).astype(jnp.bfloat16)
            rdmas[d] = make_rdma(d)
            rdmas[d].start()

        w_blk = w_ref[:, pl.ds(my * n_per, n_per)]
        yblk = jnp.dot(xv, w_blk, preferred_element_type=jnp.float32)
        out_ref[pl.ds(my * m_per, m_per), :] = yblk * jax.nn.sigmoid(yblk)

        for d in (1, 3, 2):
            rdmas[d].wait_recv()
            src = (my - d) % N_DEV
            out_ref[pl.ds(src * m_per, m_per), :] = recv_ref[
                d - 1, :, :
            ].astype(jnp.float32)
        for d in (1, 3, 2):
            rdmas[d].wait_send()

    return pl.pallas_call(
        body,
        out_shape=jax.ShapeDtypeStruct((m, n_per), jnp.float32),
        in_specs=[
            pl.BlockSpec(memory_space=pltpu.VMEM),
            pl.BlockSpec(memory_space=pltpu.VMEM),
        ],
        out_specs=pl.BlockSpec(memory_space=pltpu.VMEM),
        scratch_shapes=[
            pltpu.VMEM((N_DEV - 1, m_per, n_per), jnp.bfloat16),
            pltpu.VMEM((N_DEV - 1, m_per, n_per), jnp.bfloat16),
            pltpu.SemaphoreType.DMA((N_DEV - 1,)),
            pltpu.SemaphoreType.DMA((N_DEV - 1,)),
        ],
        compiler_params=pltpu.CompilerParams(collective_id=0),
    )(x, w_mat)


# device time: 27417 ns/iter; 1.4958x vs baseline; 1.0193x over previous
import jax
import jax.numpy as jnp
from jax import lax
from jax.experimental import pallas as pl
from jax.experimental.pallas import tpu as pltpu

N_DEV = 4


def kernel(x, w_mat):
    m_per, k = x.shape
    n = w_mat.shape[1]
    n_per = n // N_DEV
    m = N_DEV * m_per

    def body(x_ref, w_ref, out_ref, comm_ref, recv_ref, send_sems, recv_sems):
        my = lax.axis_index("i")

        barrier_sem = pltpu.get_barrier_semaphore()
        for d in range(1, N_DEV):
            pl.semaphore_signal(
                barrier_sem, inc=1,
                device_id=((my + d) % N_DEV,),
                device_id_type=pl.DeviceIdType.MESH,
            )
        xv = x_ref[:, :]

        def make_rdma(d):
            tgt = (my + d) % N_DEV
            return pltpu.make_async_remote_copy(
                src_ref=comm_ref.at[d - 1],
                dst_ref=recv_ref.at[d - 1],
                send_sem=send_sems.at[d - 1],
                recv_sem=recv_sems.at[d - 1],
                device_id=(tgt,),
                device_id_type=pl.DeviceIdType.MESH,
            )

        def chunk(tgt):
            w_blk = w_ref[:, pl.ds(tgt * n_per, n_per)]
            yblk = jnp.dot(xv, w_blk, preferred_element_type=jnp.float32)
            return yblk * jax.nn.sigmoid(yblk)

        rdmas = {}
        for d in (2, 1, 3):
            comm_ref[d - 1, :, :] = chunk((my + d) % N_DEV).astype(jnp.bfloat16)
            if d == 2:
                pl.semaphore_wait(barrier_sem, N_DEV - 1)
            rdmas[d] = make_rdma(d)
            rdmas[d].start()

        w_blk = w_ref[:, pl.ds(my * n_per, n_per)]
        yblk = jnp.dot(xv, w_blk, preferred_element_type=jnp.float32)
        out_ref[pl.ds(my * m_per, m_per), :] = yblk * jax.nn.sigmoid(yblk)

        for d in (1, 3, 2):
            rdmas[d].wait_recv()
            src = (my - d) % N_DEV
            out_ref[pl.ds(src * m_per, m_per), :] = recv_ref[
                d - 1, :, :
            ].astype(jnp.float32)
        for d in (1, 3, 2):
            rdmas[d].wait_send()

    return pl.pallas_call(
        body,
        out_shape=jax.ShapeDtypeStruct((m, n_per), jnp.float32),
        in_specs=[
            pl.BlockSpec(memory_space=pltpu.VMEM),
            pl.BlockSpec(memory_space=pltpu.VMEM),
        ],
        out_specs=pl.BlockSpec(memory_space=pltpu.VMEM),
        scratch_shapes=[
            pltpu.VMEM((N_DEV - 1, m_per, n_per), jnp.bfloat16),
            pltpu.VMEM((N_DEV - 1, m_per, n_per), jnp.bfloat16),
            pltpu.SemaphoreType.DMA((N_DEV - 1,)),
            pltpu.SemaphoreType.DMA((N_DEV - 1,)),
        ],
        compiler_params=pltpu.CompilerParams(collective_id=0),
    )(x, w_mat)
